# baseline (speedup 1.0000x reference)
import functools

import jax
import jax.numpy as jnp
from jax import lax
from jax.experimental import pallas as pl
from jax.experimental.pallas import tpu as pltpu
from jax.experimental.pallas import tpu_sc as plsc

_LANES = 16


def _table_body(d, w_ref, x_ref, b_ref, out_ref):
    w = w_ref[...]
    w4 = jnp.concatenate([w[:, :d], w[:, d:]], axis=0)
    b4 = jnp.concatenate([b_ref[...], jnp.zeros_like(b_ref[...])], axis=0)
    out_ref[...] = (
        lax.dot_general(
            w4, x_ref[...],
            (((1,), (1,)), ((), ())),
            preferred_element_type=jnp.float32,
        )
        + b4
    )


def _edge_body(epw, wmax, n_nodes, n_cores, tab_hbm, edge_hbm, out_hbm,
               tab_sh, tab_v, idx_v, out_v, tab_sem, idx_sem, out_sem):
    wid = lax.axis_index("s") * n_cores + lax.axis_index("c")
    a = wid * epw // 128 * 128
    cnt = (wid + 1) * epw // 128 * 128 - a
    body = wmax - 128
    half = body // 256 * 128

    cp_idx = pltpu.async_copy(edge_hbm.at[:, pl.ds(a, wmax)], idx_v, idx_sem)
    with jax.named_scope("tab_spmem"):
        @pl.when(lax.axis_index("s") == 0)
        def _stage():
            pltpu.sync_copy(tab_hbm, tab_sh)
        plsc.subcore_barrier()
    with jax.named_scope("in_dma"):
        cp_tab = pltpu.async_copy(tab_sh, tab_v, tab_sem)
        cp_tab.wait()
        cp_idx.wait()

    n1, n2, n3 = n_nodes, 2 * n_nodes, 3 * n_nodes

    def chunk(off):
        s = idx_v[0, pl.ds(off, _LANES)]
        d = idx_v[1, pl.ds(off, _LANES)]
        a0 = plsc.load_gather(tab_v, [s])
        a1 = plsc.load_gather(tab_v, [s + n1])
        b0 = plsc.load_gather(tab_v, [d + n2])
        b1 = plsc.load_gather(tab_v, [d + n3])
        out_v[0, pl.ds(off, _LANES)] = a0 + b0
        out_v[1, pl.ds(off, _LANES)] = a1 + b1

    with jax.named_scope("loop1"):
        plsc.parallel_loop(0, half, _LANES, unroll=4)(chunk)
    cp_out = pltpu.async_copy(
        out_v.at[:, pl.ds(0, half)], out_hbm.at[:, pl.ds(a, half)], out_sem
    )
    with jax.named_scope("loop2"):
        plsc.parallel_loop(half, cnt, _LANES, unroll=4)(chunk)
    pltpu.sync_copy(
        out_v.at[:, pl.ds(half, body - half)],
        out_hbm.at[:, pl.ds(a + half, body - half)],
    )

    @pl.when(cnt == wmax)
    def _tail():
        pltpu.sync_copy(
            out_v.at[:, pl.ds(body, 128)], out_hbm.at[:, pl.ds(a + body, 128)]
        )

    with jax.named_scope("out_drain"):
        cp_out.wait()


def kernel(x, edge_index, W, b):
    n_nodes, d = x.shape
    n_classes = W.shape[0]
    n_edges = edge_index.shape[1]
    assert n_classes == 2 and W.shape[1] == 2 * d and n_edges % 128 == 0

    blk = 1024
    grid = -(-n_nodes // blk)
    table = pl.pallas_call(
        functools.partial(_table_body, d),
        grid=(grid,),
        in_specs=[
            pl.BlockSpec((n_classes, 2 * d), lambda i: (0, 0)),
            pl.BlockSpec((blk, d), lambda i: (i, 0)),
            pl.BlockSpec((n_classes, 1), lambda i: (0, 0)),
        ],
        out_specs=pl.BlockSpec((2 * n_classes, blk), lambda i: (0, i)),
        out_shape=jax.ShapeDtypeStruct((2 * n_classes, n_nodes), jnp.float32),
        compiler_params=pltpu.CompilerParams(
            dimension_semantics=("arbitrary",)
        ),
    )(W, x, b.reshape(n_classes, 1))

    info = plsc.get_sparse_core_info()
    n_workers = info.num_cores * info.num_subcores
    epw = n_edges // n_workers
    cnts = {((w + 1) * epw // 128 - w * epw // 128) * 128
            for w in range(n_workers)}
    wmax = max(cnts)
    assert cnts <= {wmax, wmax - 128} and wmax % _LANES == 0

    mesh = plsc.VectorSubcoreMesh(core_axis_name="c", subcore_axis_name="s")
    edge_fn = pl.kernel(
        functools.partial(_edge_body, epw, wmax, n_nodes, info.num_cores),
        out_type=jax.ShapeDtypeStruct((n_classes, n_edges), jnp.float32),
        mesh=mesh,
        compiler_params=pltpu.CompilerParams(needs_layout_passes=False),
        scratch_types=[
            pltpu.VMEM_SHARED((n_nodes * 2 * n_classes,), jnp.float32),
            pltpu.VMEM((n_nodes * 2 * n_classes,), jnp.float32),
            pltpu.VMEM((2, wmax), jnp.int32),
            pltpu.VMEM((n_classes, wmax), jnp.float32),
            pltpu.SemaphoreType.DMA,
            pltpu.SemaphoreType.DMA,
            pltpu.SemaphoreType.DMA,
        ],
    )
    out = edge_fn(table.reshape(-1), edge_index)
    return out.T


# speedup vs baseline: 1.1216x; 1.1216x over previous
import functools

import jax
import jax.numpy as jnp
from jax import lax
from jax.experimental import pallas as pl
from jax.experimental.pallas import tpu as pltpu
from jax.experimental.pallas import tpu_sc as plsc

_LANES = 16


def _table_body(d, w_ref, x_ref, b_ref, out_ref):
    w = w_ref[...]
    w4 = jnp.concatenate([w[:, :d], w[:, d:]], axis=0)
    b4 = jnp.concatenate([b_ref[...], jnp.zeros_like(b_ref[...])], axis=0)
    out_ref[...] = (
        lax.dot_general(
            w4, x_ref[...],
            (((1,), (1,)), ((), ())),
            preferred_element_type=jnp.float32,
        )
        + b4
    )


def _edge_body(epw, wmax, n_nodes, n_cores, tab_hbm, edge_hbm, out_hbm,
               tab_sh, tab_v, idx_v, out_v, tab_sem, idx_sem, out_sem):
    wid = lax.axis_index("s") * n_cores + lax.axis_index("c")
    a = wid * epw // 128 * 128
    cnt = (wid + 1) * epw // 128 * 128 - a
    body = wmax - 128
    half = body // 256 * 128

    cp_idx = pltpu.async_copy(edge_hbm.at[:, pl.ds(a, wmax)], idx_v, idx_sem)
    with jax.named_scope("tab_spmem"):
        @pl.when(lax.axis_index("s") == 0)
        def _stage():
            pltpu.sync_copy(tab_hbm, tab_sh)
        plsc.subcore_barrier()
    with jax.named_scope("in_dma"):
        cp_tab = pltpu.async_copy(tab_sh, tab_v, tab_sem)
        cp_tab.wait()
        cp_idx.wait()

    n1, n2, n3 = n_nodes, 2 * n_nodes, 3 * n_nodes

    def chunk(off):
        s = idx_v[0, pl.ds(off, _LANES)]
        d = idx_v[1, pl.ds(off, _LANES)]
        a0 = plsc.load_gather(tab_v, [s])
        a1 = plsc.load_gather(tab_v, [s + n1])
        b0 = plsc.load_gather(tab_v, [d + n2])
        b1 = plsc.load_gather(tab_v, [d + n3])
        out_v[0, pl.ds(off, _LANES)] = a0 + b0
        out_v[1, pl.ds(off, _LANES)] = a1 + b1

    with jax.named_scope("loop1"):
        plsc.parallel_loop(0, half, _LANES, unroll=4)(chunk)
    cp_out = pltpu.async_copy(
        out_v.at[:, pl.ds(0, half)], out_hbm.at[:, pl.ds(a, half)], out_sem
    )
    with jax.named_scope("loop2"):
        plsc.parallel_loop(half, cnt, _LANES, unroll=4)(chunk)
    pltpu.sync_copy(
        out_v.at[:, pl.ds(half, body - half)],
        out_hbm.at[:, pl.ds(a + half, body - half)],
    )

    @pl.when(cnt == wmax)
    def _tail():
        pltpu.sync_copy(
            out_v.at[:, pl.ds(body, 128)], out_hbm.at[:, pl.ds(a + body, 128)]
        )

    with jax.named_scope("out_drain"):
        cp_out.wait()


def kernel(x, edge_index, W, b):
    n_nodes, d = x.shape
    n_classes = W.shape[0]
    n_edges = edge_index.shape[1]
    assert n_classes == 2 and W.shape[1] == 2 * d and n_edges % 128 == 0

    table = pl.pallas_call(
        functools.partial(_table_body, d),
        out_shape=jax.ShapeDtypeStruct((2 * n_classes, n_nodes), jnp.float32),
    )(W, x, b.reshape(n_classes, 1))

    info = plsc.get_sparse_core_info()
    n_workers = info.num_cores * info.num_subcores
    epw = n_edges // n_workers
    cnts = {((w + 1) * epw // 128 - w * epw // 128) * 128
            for w in range(n_workers)}
    wmax = max(cnts)
    assert cnts <= {wmax, wmax - 128} and wmax % _LANES == 0

    mesh = plsc.VectorSubcoreMesh(core_axis_name="c", subcore_axis_name="s")
    edge_fn = pl.kernel(
        functools.partial(_edge_body, epw, wmax, n_nodes, info.num_cores),
        out_type=jax.ShapeDtypeStruct((n_classes, n_edges), jnp.float32),
        mesh=mesh,
        compiler_params=pltpu.CompilerParams(needs_layout_passes=False),
        scratch_types=[
            pltpu.VMEM_SHARED((n_nodes * 2 * n_classes,), jnp.float32),
            pltpu.VMEM((n_nodes * 2 * n_classes,), jnp.float32),
            pltpu.VMEM((2, wmax), jnp.int32),
            pltpu.VMEM((n_classes, wmax), jnp.float32),
            pltpu.SemaphoreType.DMA,
            pltpu.SemaphoreType.DMA,
            pltpu.SemaphoreType.DMA,
        ],
    )
    out = edge_fn(table.reshape(-1), edge_index)
    return out.T
